# baseline (speedup 1.0000x reference)
import functools

import jax
import jax.numpy as jnp
from jax import lax
from jax.experimental import pallas as pl
from jax.experimental.pallas import tpu as pltpu
from jax.experimental.pallas import tpu_sc as plsc

ORG_VOCAB = 100000
EXTRA_VOCAB = 256
FULL_VOCAB = ORG_VOCAB + EXTRA_VOCAB
D = 1024
R = 16
NLORA = 8
T = 16384

NCHUNK = 4
TH = T // NCHUNK
NC = 2
NS = 16
NW = NC * NS
TPW = TH // NW
CW = 32
NCH = TPW // CW
ACH = 128
NACH = TPW // ACH

NCB = FULL_VOCAB * NLORA // 128
HALF_STRIDE = NCB * 1024

_sc_mesh = plsc.VectorSubcoreMesh(core_axis_name="c", subcore_axis_name="s")


def _make_sc_gather_chunk(h):
    @functools.partial(
        pl.kernel,
        mesh=_sc_mesh,
        out_type=[
            jax.ShapeDtypeStruct((TH, D), jnp.float32),
            jax.ShapeDtypeStruct((R, TH), jnp.float32),
        ],
        scratch_types=[
            pltpu.VMEM((TPW,), jnp.int32),
            pltpu.VMEM((TPW,), jnp.int32),
            pltpu.VMEM((NCH, CW), jnp.int32),
            pltpu.VMEM((R * NACH, ACH), jnp.int32),
            pltpu.VMEM((3, CW, D), jnp.float32),
            pltpu.VMEM((R, TPW), jnp.float32),
            pltpu.SemaphoreType.DMA((3,)),
            pltpu.SemaphoreType.DMA((3,)),
            pltpu.SemaphoreType.DMA,
        ],
    )
    def _sc_gather(x_hbm, li_hbm, w_hbm, laf_hbm, base_hbm, at_hbm,
                   x_v, li_v, widx, aidx, rowbuf, abuf, gsem, ssem, asem):
        wid = lax.axis_index("s") * NC + lax.axis_index("c")
        tok0 = wid * TPW
        src0 = h * TH + tok0

        pltpu.sync_copy(x_hbm.at[pl.ds(src0, TPW)], x_v)
        pltpu.sync_copy(li_hbm.at[pl.ds(src0, TPW)], li_v)

        for i in range(TPW // 16):
            xv = x_v[pl.ds(i * 16, 16)]
            lv = li_v[pl.ds(i * 16, 16)]
            widx[i // 2, pl.ds((i % 2) * 16, 16)] = xv
            cv = xv + lv * FULL_VOCAB
            bv = ((cv >> 7) << 10) + (cv & 127)
            for r in range(R):
                off = (r // 8) * HALF_STRIDE + (r % 8) * 128
                aidx[r * NACH + i // 8, pl.ds((i % 8) * 16, 16)] = bv + off

        a_copies = []
        for r in range(R):
            for j in range(NACH):
                a_copies.append(pltpu.async_copy(
                    laf_hbm.at[aidx.at[r * NACH + j]],
                    abuf.at[r, pl.ds(j * ACH, ACH)],
                    asem))

        gathers = {}
        writes = {}
        gathers[0] = pltpu.async_copy(w_hbm.at[widx.at[0]], rowbuf.at[0],
                                      gsem.at[0])
        gathers[1] = pltpu.async_copy(w_hbm.at[widx.at[1]], rowbuf.at[1],
                                      gsem.at[1])
        for j in range(NCH):
            if j + 2 < NCH:
                if j - 1 >= 0:
                    writes[j - 1].wait()
                gathers[j + 2] = pltpu.async_copy(
                    w_hbm.at[widx.at[j + 2]], rowbuf.at[(j + 2) % 3],
                    gsem.at[(j + 2) % 3])
            gathers[j].wait()
            writes[j] = pltpu.async_copy(
                rowbuf.at[j % 3], base_hbm.at[pl.ds(tok0 + j * CW, CW)],
                ssem.at[j % 3])

        for c in a_copies:
            c.wait()
        pltpu.sync_copy(abuf, at_hbm.at[:, pl.ds(tok0, TPW)])
        for j in range(max(0, NCH - 3), NCH):
            writes[j].wait()

    return _sc_gather


_sc_gather_h = [_make_sc_gather_chunk(h) for h in range(NCHUNK)]

TB = 2048
NBH = TH // TB


def _combine_body(base_ref, at_ref, li_ref, bt_ref, o_ref):
    a8 = jnp.concatenate([at_ref[...]] * NLORA, axis=0)
    grp = lax.broadcasted_iota(jnp.int32, (NLORA * R, TB), 0) // R
    am = jnp.where(grp == li_ref[...], a8, 0.0)
    o_ref[...] = base_ref[...] + lax.dot_general(
        am, bt_ref[...], (((0,), (0,)), ((), ())),
        preferred_element_type=jnp.float32)


def _combine_body_carrier(carrier_ref, base_ref, at_ref, li_ref, bt_ref, o_ref):
    del carrier_ref
    _combine_body(base_ref, at_ref, li_ref, bt_ref, o_ref)


def kernel(x, lora_indices, weight, lora_a_stacked_2d, lora_b_stacked):
    laf = (lora_a_stacked_2d.T
           .reshape(2, 8, NCB, 128)
           .transpose(0, 2, 1, 3)
           .reshape(2 * NCB * 8 * 128))
    bt = lora_b_stacked[:, 0].transpose(0, 2, 1).reshape(NLORA * R, D)
    li2 = lora_indices.reshape(1, T)

    parts = [_sc_gather_h[h](x, lora_indices, weight, laf)
             for h in range(NCHUNK)]

    out = None
    for h in range(NCHUNK):
        base_h, at_h = parts[h]
        if h == 0:
            out = pl.pallas_call(
                _combine_body,
                grid=(NBH,),
                in_specs=[
                    pl.BlockSpec((TB, D), lambda i: (i, 0)),
                    pl.BlockSpec((R, TB), lambda i: (0, i)),
                    pl.BlockSpec((1, TB), lambda i: (0, i)),
                    pl.BlockSpec((NLORA * R, D), lambda i: (0, 0)),
                ],
                out_specs=pl.BlockSpec((TB, D), lambda i: (i, 0)),
                out_shape=jax.ShapeDtypeStruct((T, D), jnp.float32),
            )(base_h, at_h, li2, bt)
        else:
            out = pl.pallas_call(
                _combine_body_carrier,
                grid=(NBH,),
                in_specs=[
                    pl.BlockSpec((8, 128), lambda i: (0, 0)),
                    pl.BlockSpec((TB, D), lambda i: (i, 0)),
                    pl.BlockSpec((R, TB), lambda i: (0, i)),
                    pl.BlockSpec((1, TB), lambda i, h=h: (0, i + h * NBH)),
                    pl.BlockSpec((NLORA * R, D), lambda i: (0, 0)),
                ],
                out_specs=pl.BlockSpec((TB, D), lambda i, h=h: (i + h * NBH, 0)),
                out_shape=jax.ShapeDtypeStruct((T, D), jnp.float32),
                input_output_aliases={0: 0},
            )(out, base_h, at_h, li2, bt)
    return out


# speedup vs baseline: 1.0033x; 1.0033x over previous
import functools

import jax
import jax.numpy as jnp
from jax import lax
from jax.experimental import pallas as pl
from jax.experimental.pallas import tpu as pltpu
from jax.experimental.pallas import tpu_sc as plsc

ORG_VOCAB = 100000
EXTRA_VOCAB = 256
FULL_VOCAB = ORG_VOCAB + EXTRA_VOCAB
D = 1024
R = 16
NLORA = 8
T = 16384

SIZES = (12288, 4096)
OFFS = (0, 12288)
NCHUNK = len(SIZES)
NC = 2
NS = 16
NW = NC * NS
CW = 32
ACH = 128

NCB = FULL_VOCAB * NLORA // 128
HALF_STRIDE = NCB * 1024

_sc_mesh = plsc.VectorSubcoreMesh(core_axis_name="c", subcore_axis_name="s")


def _make_sc_gather_chunk(h):
    th = SIZES[h]
    chunk_off = OFFS[h]
    TPW = th // NW
    NCH = TPW // CW
    NACH = TPW // ACH

    @functools.partial(
        pl.kernel,
        mesh=_sc_mesh,
        out_type=[
            jax.ShapeDtypeStruct((th, D), jnp.float32),
            jax.ShapeDtypeStruct((R, th), jnp.float32),
        ],
        scratch_types=[
            pltpu.VMEM((TPW,), jnp.int32),
            pltpu.VMEM((TPW,), jnp.int32),
            pltpu.VMEM((NCH, CW), jnp.int32),
            pltpu.VMEM((R * NACH, ACH), jnp.int32),
            pltpu.VMEM((3, CW, D), jnp.float32),
            pltpu.VMEM((R, TPW), jnp.float32),
            pltpu.SemaphoreType.DMA((3,)),
            pltpu.SemaphoreType.DMA((3,)),
            pltpu.SemaphoreType.DMA,
        ],
    )
    def _sc_gather(x_hbm, li_hbm, w_hbm, laf_hbm, base_hbm, at_hbm,
                   x_v, li_v, widx, aidx, rowbuf, abuf, gsem, ssem, asem):
        wid = lax.axis_index("s") * NC + lax.axis_index("c")
        tok0 = wid * TPW
        src0 = chunk_off + tok0

        pltpu.sync_copy(x_hbm.at[pl.ds(src0, TPW)], x_v)
        pltpu.sync_copy(li_hbm.at[pl.ds(src0, TPW)], li_v)

        for i in range(TPW // 16):
            xv = x_v[pl.ds(i * 16, 16)]
            lv = li_v[pl.ds(i * 16, 16)]
            widx[i // 2, pl.ds((i % 2) * 16, 16)] = xv
            cv = xv + lv * FULL_VOCAB
            bv = ((cv >> 7) << 10) + (cv & 127)
            for r in range(R):
                off = (r // 8) * HALF_STRIDE + (r % 8) * 128
                aidx[r * NACH + i // 8, pl.ds((i % 8) * 16, 16)] = bv + off

        a_copies = []
        for r in range(R):
            for j in range(NACH):
                a_copies.append(pltpu.async_copy(
                    laf_hbm.at[aidx.at[r * NACH + j]],
                    abuf.at[r, pl.ds(j * ACH, ACH)],
                    asem))

        gathers = {}
        writes = {}
        gathers[0] = pltpu.async_copy(w_hbm.at[widx.at[0]], rowbuf.at[0],
                                      gsem.at[0])
        gathers[1] = pltpu.async_copy(w_hbm.at[widx.at[1]], rowbuf.at[1],
                                      gsem.at[1])
        for j in range(NCH):
            if j + 2 < NCH:
                if j - 1 >= 0:
                    writes[j - 1].wait()
                gathers[j + 2] = pltpu.async_copy(
                    w_hbm.at[widx.at[j + 2]], rowbuf.at[(j + 2) % 3],
                    gsem.at[(j + 2) % 3])
            gathers[j].wait()
            writes[j] = pltpu.async_copy(
                rowbuf.at[j % 3], base_hbm.at[pl.ds(tok0 + j * CW, CW)],
                ssem.at[j % 3])

        for c in a_copies:
            c.wait()
        pltpu.sync_copy(abuf, at_hbm.at[:, pl.ds(tok0, TPW)])
        for j in range(max(0, NCH - 3), NCH):
            writes[j].wait()

    return _sc_gather


_sc_gather_h = [_make_sc_gather_chunk(h) for h in range(NCHUNK)]

TB = 2048


def _combine_body(base_ref, at_ref, li_ref, bt_ref, o_ref):
    a8 = jnp.concatenate([at_ref[...]] * NLORA, axis=0)
    grp = lax.broadcasted_iota(jnp.int32, (NLORA * R, TB), 0) // R
    am = jnp.where(grp == li_ref[...], a8, 0.0)
    o_ref[...] = base_ref[...] + lax.dot_general(
        am, bt_ref[...], (((0,), (0,)), ((), ())),
        preferred_element_type=jnp.float32)


def _combine_body_carrier(carrier_ref, base_ref, at_ref, li_ref, bt_ref, o_ref):
    del carrier_ref
    _combine_body(base_ref, at_ref, li_ref, bt_ref, o_ref)


def kernel(x, lora_indices, weight, lora_a_stacked_2d, lora_b_stacked):
    laf = (lora_a_stacked_2d.T
           .reshape(2, 8, NCB, 128)
           .transpose(0, 2, 1, 3)
           .reshape(2 * NCB * 8 * 128))
    bt = lora_b_stacked[:, 0].transpose(0, 2, 1).reshape(NLORA * R, D)
    li2 = lora_indices.reshape(1, T)

    parts = [_sc_gather_h[h](x, lora_indices, weight, laf)
             for h in range(NCHUNK)]

    out = None
    for h in range(NCHUNK):
        base_h, at_h = parts[h]
        nbh = SIZES[h] // TB
        b0 = OFFS[h] // TB
        if h == 0:
            out = pl.pallas_call(
                _combine_body,
                grid=(nbh,),
                in_specs=[
                    pl.BlockSpec((TB, D), lambda i: (i, 0)),
                    pl.BlockSpec((R, TB), lambda i: (0, i)),
                    pl.BlockSpec((1, TB), lambda i: (0, i)),
                    pl.BlockSpec((NLORA * R, D), lambda i: (0, 0)),
                ],
                out_specs=pl.BlockSpec((TB, D), lambda i: (i, 0)),
                out_shape=jax.ShapeDtypeStruct((T, D), jnp.float32),
            )(base_h, at_h, li2, bt)
        else:
            out = pl.pallas_call(
                _combine_body_carrier,
                grid=(nbh,),
                in_specs=[
                    pl.BlockSpec((8, 128), lambda i: (0, 0)),
                    pl.BlockSpec((TB, D), lambda i: (i, 0)),
                    pl.BlockSpec((R, TB), lambda i: (0, i)),
                    pl.BlockSpec((1, TB), lambda i, b0=b0: (0, i + b0)),
                    pl.BlockSpec((NLORA * R, D), lambda i: (0, 0)),
                ],
                out_specs=pl.BlockSpec((TB, D), lambda i, b0=b0: (i + b0, 0)),
                out_shape=jax.ShapeDtypeStruct((T, D), jnp.float32),
                input_output_aliases={0: 0},
            )(out, base_h, at_h, li2, bt)
    return out


# speedup vs baseline: 1.0149x; 1.0116x over previous
import functools

import jax
import jax.numpy as jnp
from jax import lax
from jax.experimental import pallas as pl
from jax.experimental.pallas import tpu as pltpu
from jax.experimental.pallas import tpu_sc as plsc

ORG_VOCAB = 100000
EXTRA_VOCAB = 256
FULL_VOCAB = ORG_VOCAB + EXTRA_VOCAB
D = 1024
R = 16
NLORA = 8
T = 16384

SIZES = (8192, 8192)
OFFS = (0, 8192)
NCHUNK = len(SIZES)
NC = 2
NS = 16
NW = NC * NS
CW = 32
ACH = 128

NCB = FULL_VOCAB * NLORA // 128
HALF_STRIDE = NCB * 1024

_sc_mesh = plsc.VectorSubcoreMesh(core_axis_name="c", subcore_axis_name="s")


def _make_sc_gather_chunk(h):
    th = SIZES[h]
    chunk_off = OFFS[h]
    TPW = th // NW
    NCH = TPW // CW
    NACH = TPW // ACH

    @functools.partial(
        pl.kernel,
        mesh=_sc_mesh,
        out_type=[
            jax.ShapeDtypeStruct((th, D), jnp.float32),
            jax.ShapeDtypeStruct((R, th), jnp.float32),
        ],
        scratch_types=[
            pltpu.VMEM((TPW,), jnp.int32),
            pltpu.VMEM((TPW,), jnp.int32),
            pltpu.VMEM((NCH, CW), jnp.int32),
            pltpu.VMEM((R * NACH, ACH), jnp.int32),
            pltpu.VMEM((3, CW, D), jnp.float32),
            pltpu.VMEM((R, TPW), jnp.float32),
            pltpu.SemaphoreType.DMA((3,)),
            pltpu.SemaphoreType.DMA((3,)),
            pltpu.SemaphoreType.DMA,
        ],
    )
    def _sc_gather(x_hbm, li_hbm, w_hbm, laf_hbm, base_hbm, at_hbm,
                   x_v, li_v, widx, aidx, rowbuf, abuf, gsem, ssem, asem):
        wid = lax.axis_index("s") * NC + lax.axis_index("c")
        tok0 = wid * TPW
        src0 = chunk_off + tok0

        pltpu.sync_copy(x_hbm.at[pl.ds(src0, TPW)], x_v)
        pltpu.sync_copy(li_hbm.at[pl.ds(src0, TPW)], li_v)

        for i in range(TPW // 16):
            xv = x_v[pl.ds(i * 16, 16)]
            lv = li_v[pl.ds(i * 16, 16)]
            widx[i // 2, pl.ds((i % 2) * 16, 16)] = xv
            cv = xv + lv * FULL_VOCAB
            bv = ((cv >> 7) << 10) + (cv & 127)
            for r in range(R):
                off = (r // 8) * HALF_STRIDE + (r % 8) * 128
                aidx[r * NACH + i // 8, pl.ds((i % 8) * 16, 16)] = bv + off

        a_copies = []
        for r in range(R):
            for j in range(NACH):
                a_copies.append(pltpu.async_copy(
                    laf_hbm.at[aidx.at[r * NACH + j]],
                    abuf.at[r, pl.ds(j * ACH, ACH)],
                    asem))

        gathers = {}
        writes = {}
        gathers[0] = pltpu.async_copy(w_hbm.at[widx.at[0]], rowbuf.at[0],
                                      gsem.at[0])
        gathers[1] = pltpu.async_copy(w_hbm.at[widx.at[1]], rowbuf.at[1],
                                      gsem.at[1])
        for j in range(NCH):
            if j + 2 < NCH:
                if j - 1 >= 0:
                    writes[j - 1].wait()
                gathers[j + 2] = pltpu.async_copy(
                    w_hbm.at[widx.at[j + 2]], rowbuf.at[(j + 2) % 3],
                    gsem.at[(j + 2) % 3])
            gathers[j].wait()
            writes[j] = pltpu.async_copy(
                rowbuf.at[j % 3], base_hbm.at[pl.ds(tok0 + j * CW, CW)],
                ssem.at[j % 3])

        for c in a_copies:
            c.wait()
        pltpu.sync_copy(abuf, at_hbm.at[:, pl.ds(tok0, TPW)])
        for j in range(max(0, NCH - 3), NCH):
            writes[j].wait()

    return _sc_gather


_sc_gather_h = [_make_sc_gather_chunk(h) for h in range(NCHUNK)]

TB = 2048


def _combine_body(base_ref, at_ref, li_ref, bt_ref, o_ref):
    a8 = jnp.concatenate([at_ref[...]] * NLORA, axis=0)
    grp = lax.broadcasted_iota(jnp.int32, (NLORA * R, TB), 0) // R
    am = jnp.where(grp == li_ref[...], a8, 0.0)
    o_ref[...] = base_ref[...] + lax.dot_general(
        am, bt_ref[...], (((0,), (0,)), ((), ())),
        preferred_element_type=jnp.float32)


def _combine_body_carrier(carrier_ref, base_ref, at_ref, li_ref, bt_ref, o_ref):
    del carrier_ref
    _combine_body(base_ref, at_ref, li_ref, bt_ref, o_ref)


def kernel(x, lora_indices, weight, lora_a_stacked_2d, lora_b_stacked):
    laf = (lora_a_stacked_2d.T
           .reshape(2, 8, NCB, 128)
           .transpose(0, 2, 1, 3)
           .reshape(2 * NCB * 8 * 128))
    bt = lora_b_stacked[:, 0].transpose(0, 2, 1).reshape(NLORA * R, D)
    li2 = lora_indices.reshape(1, T)

    parts = [_sc_gather_h[h](x, lora_indices, weight, laf)
             for h in range(NCHUNK)]

    out = None
    for h in range(NCHUNK):
        base_h, at_h = parts[h]
        nbh = SIZES[h] // TB
        b0 = OFFS[h] // TB
        if h == 0:
            out = pl.pallas_call(
                _combine_body,
                grid=(nbh,),
                in_specs=[
                    pl.BlockSpec((TB, D), lambda i: (i, 0)),
                    pl.BlockSpec((R, TB), lambda i: (0, i)),
                    pl.BlockSpec((1, TB), lambda i: (0, i)),
                    pl.BlockSpec((NLORA * R, D), lambda i: (0, 0)),
                ],
                out_specs=pl.BlockSpec((TB, D), lambda i: (i, 0)),
                out_shape=jax.ShapeDtypeStruct((T, D), jnp.float32),
            )(base_h, at_h, li2, bt)
        else:
            out = pl.pallas_call(
                _combine_body_carrier,
                grid=(nbh,),
                in_specs=[
                    pl.BlockSpec((8, 128), lambda i: (0, 0)),
                    pl.BlockSpec((TB, D), lambda i: (i, 0)),
                    pl.BlockSpec((R, TB), lambda i: (0, i)),
                    pl.BlockSpec((1, TB), lambda i, b0=b0: (0, i + b0)),
                    pl.BlockSpec((NLORA * R, D), lambda i: (0, 0)),
                ],
                out_specs=pl.BlockSpec((TB, D), lambda i, b0=b0: (i + b0, 0)),
                out_shape=jax.ShapeDtypeStruct((T, D), jnp.float32),
                input_output_aliases={0: 0},
            )(out, base_h, at_h, li2, bt)
    return out
